# Initial kernel scaffold; baseline (speedup 1.0000x reference)
import jax
import jax.numpy as jnp
from jax.experimental import pallas as pl


def kernel(inp, history, nodes_min, nodes_extent, table):
    raise NotImplementedError("write your pallas kernel here")


# speedup vs baseline: 2.5942x; 2.5942x over previous
import functools

import jax
import jax.numpy as jnp
from jax import lax
from jax.experimental import pallas as pl
from jax.experimental.pallas import tpu as pltpu
from jax.experimental.pallas import tpu_sc as plsc

N_RAYS = 65536
N_POINTS = 4
ENC_DIM = 8
ENC_DEPTH = 8
TABLE_SIZE = 2097152
N_NODES = 1000000
OUT_COLS = ENC_DEPTH * N_POINTS * ENC_DIM

_PIS = (774363409, 2654435761, 805459861, 100000007,
        334363391, 1334363413, 734363407, 2134363393)
_HASH_C = tuple(((j + 1) * _PIS[j]) % TABLE_SIZE for j in range(8))

NC, NS = 2, 16
NW = NC * NS
RPW = N_RAYS // NW
B = 64
NBLK = RPW // B
NGRP = B // 16
CHUNK = 128
NSLOT = B * ENC_DEPTH


def _i32(x):
  return jnp.int32(x)


def _sc_body(inp_hbm, hist_hbm, nmin_hbm, next_hbm, table_hbm, out_hbm,
             hist_v, tidx_v, ridx_v, ridx1_v, feat_v,
             mnA_v, mnB_v, exA_v, exB_v, inp_v, out_v, sem):
  wid = lax.axis_index("s") * _i32(NC) + lax.axis_index("c")
  ray0 = wid * _i32(RPW)
  iota = lax.iota(jnp.int32, 16)
  iota8 = iota * _i32(8)
  col_ids = [jnp.full((16,), c, jnp.int32) for c in range(ENC_DIM)]
  zero16 = jnp.zeros((16,), jnp.int32)
  f_one = jnp.float32(1.0)
  f_zero = jnp.float32(0.0)

  def block_body(b, carry):
    rb = ray0 + b * _i32(B)
    pltpu.sync_copy(hist_hbm.at[pl.ds(rb * _i32(ENC_DEPTH), B * ENC_DEPTH)],
                    hist_v)
    pltpu.sync_copy(inp_hbm.at[pl.ds(rb, B)], inp_v)

    def gen_body(i, carry2):
      for g in range(NGRP):
        r0 = g * 16
        hv = plsc.load_gather(hist_v, [iota8 + (_i32(r0 * ENC_DEPTH) + i)])
        for j in range(8):
          plsc.store_scatter(
              tidx_v,
              [iota + ((i * _i32(8) + _i32(j)) * _i32(B) + _i32(r0))],
              hv ^ _i32(_HASH_C[j]))
        rm = (hv * _i32(3)) >> _i32(3)
        slot = iota + (i * _i32(NGRP * 16) + _i32(r0))
        plsc.store_scatter(ridx_v, [slot], rm)
        plsc.store_scatter(ridx1_v, [slot],
                           jnp.minimum(rm + _i32(1), _i32(N_NODES * 3 // 8 - 1)))
      return carry2
    lax.fori_loop(_i32(0), _i32(ENC_DEPTH), gen_body, _i32(0))

    copies = []
    for c in range(B * 64 // CHUNK):
      copies.append(pltpu.async_copy(
          table_hbm.at[tidx_v.at[pl.ds(c * CHUNK, CHUNK)]],
          feat_v.at[pl.ds(c * CHUNK, CHUNK), :], sem))
    for c in range(NSLOT // CHUNK):
      sl = pl.ds(c * CHUNK, CHUNK)
      copies.append(pltpu.async_copy(
          nmin_hbm.at[ridx_v.at[sl]], mnA_v.at[sl, :], sem))
      copies.append(pltpu.async_copy(
          nmin_hbm.at[ridx1_v.at[sl]], mnB_v.at[sl, :], sem))
      copies.append(pltpu.async_copy(
          next_hbm.at[ridx_v.at[sl]], exA_v.at[sl, :], sem))
      copies.append(pltpu.async_copy(
          next_hbm.at[ridx1_v.at[sl]], exB_v.at[sl, :], sem))
    for cp in copies:
      cp.wait()

    for g in range(NGRP):
      r0 = g * 16
      rays = iota + _i32(r0)
      pts = [plsc.load_gather(inp_v, [rays, col_ids[p], col_ids[c]])
             for p in range(N_POINTS) for c in range(3)]

      def depth_body(i, carry2, r0=r0, rays=rays, pts=pts):
        hv = plsc.load_gather(hist_v, [iota8 + (_i32(r0 * ENC_DEPTH) + i)])
        rem = (hv * _i32(3)) & _i32(7)
        slot = iota + (i * _i32(NGRP * 16) + _i32(r0))
        mn, rex = [], []
        for c in range(3):
          colc = rem + _i32(c)
          colm = colc & _i32(7)
          is_lo = colc < _i32(8)
          mn.append(jnp.where(is_lo,
                              plsc.load_gather(mnA_v, [slot, colm]),
                              plsc.load_gather(mnB_v, [slot, colm])))
          ex = jnp.where(is_lo,
                         plsc.load_gather(exA_v, [slot, colm]),
                         plsc.load_gather(exB_v, [slot, colm]))
          rex.append(f_one / ex)
        ws = []
        for p in range(N_POINTS):
          xd, yd, zd = (
              jnp.clip((pts[p * 3 + c] - mn[c]) * rex[c], f_zero, f_one)
              for c in range(3))
          x0, y0, z0 = f_one - xd, f_one - yd, f_one - zd
          a00, a10, a01, a11 = x0 * y0, xd * y0, x0 * yd, xd * yd
          ws.append((a00 * z0, a10 * z0, a01 * z0, a00 * zd,
                     a10 * zd, a01 * zd, a11 * z0, a11 * zd))
        for d in range(ENC_DIM):
          fv = [plsc.load_gather(
                    feat_v,
                    [iota + ((i * _i32(8) + _i32(j)) * _i32(B) + _i32(r0)),
                     col_ids[d]])
                for j in range(8)]
          for p in range(N_POINTS):
            acc = ws[p][0] * fv[0]
            for j in range(1, 8):
              acc = acc + ws[p][j] * fv[j]
            plsc.store_scatter(
                out_v, [rays, zero16 + (i * _i32(32) + _i32(p * 8 + d))],
                acc)
        return carry2
      lax.fori_loop(_i32(0), _i32(ENC_DEPTH), depth_body, _i32(0))

    pltpu.sync_copy(out_v, out_hbm.at[pl.ds(rb, B), :])
    return carry
  lax.fori_loop(_i32(0), _i32(NBLK), block_body, _i32(0))


@functools.partial(
    pl.kernel,
    out_type=jax.ShapeDtypeStruct((N_RAYS, OUT_COLS), jnp.float32),
    mesh=plsc.VectorSubcoreMesh(core_axis_name="c", subcore_axis_name="s",
                                num_cores=NC, num_subcores=NS),
    scratch_types=[
        pltpu.VMEM((B * ENC_DEPTH,), jnp.int32),
        pltpu.VMEM((B * 64,), jnp.int32),
        pltpu.VMEM((NSLOT,), jnp.int32),
        pltpu.VMEM((NSLOT,), jnp.int32),
        pltpu.VMEM((B * 64, ENC_DIM), jnp.float32),
        pltpu.VMEM((NSLOT, 8), jnp.float32),
        pltpu.VMEM((NSLOT, 8), jnp.float32),
        pltpu.VMEM((NSLOT, 8), jnp.float32),
        pltpu.VMEM((NSLOT, 8), jnp.float32),
        pltpu.VMEM((B, N_POINTS, 3), jnp.float32),
        pltpu.VMEM((B, OUT_COLS), jnp.float32),
        pltpu.SemaphoreType.DMA,
    ],
    compiler_params=pltpu.CompilerParams(needs_layout_passes=False,
                                         use_tc_tiling_on_sc=False),
)
def _nbvh_sc(*refs):
  _sc_body(*refs)


def kernel(inp, history, nodes_min, nodes_extent, table):
  hist_i = history.astype(jnp.int32).reshape(-1)
  nmin8 = nodes_min.astype(jnp.float32).reshape(N_NODES * 3 // 8, 8)
  next8 = nodes_extent.astype(jnp.float32).reshape(N_NODES * 3 // 8, 8)
  return _nbvh_sc(inp.astype(jnp.float32), hist_i, nmin8, next8,
                  table.astype(jnp.float32))


# speedup vs baseline: 8.7995x; 3.3920x over previous
import functools

import jax
import jax.numpy as jnp
from jax import lax
from jax.experimental import pallas as pl
from jax.experimental.pallas import tpu as pltpu
from jax.experimental.pallas import tpu_sc as plsc

N_RAYS = 65536
N_POINTS = 4
ENC_DIM = 8
ENC_DEPTH = 8
TABLE_SIZE = 2097152
N_NODES = 1000000
OUT_COLS = ENC_DEPTH * N_POINTS * ENC_DIM

_PIS = (774363409, 2654435761, 805459861, 100000007,
        334363391, 1334363413, 734363407, 2134363393)
_HASH_C = tuple(((j + 1) * _PIS[j]) % TABLE_SIZE for j in range(8))

NC, NS = 2, 16
NW = NC * NS
RPW = N_RAYS // NW
B = 64
NBLK = RPW // B
NGRP = B // 16
CHUNK = 128
NSLOT = B * ENC_DEPTH


def _i32(x):
  return jnp.int32(x)


def _sc_body(inp_hbm, hist_hbm, nodes_hbm, table_hbm, out_hbm,
             hist_v, tidx_v, feat_v, node_v, inp_v, out_v, sem):
  wid = lax.axis_index("s") * _i32(NC) + lax.axis_index("c")
  ray0 = wid * _i32(RPW)
  iota = lax.iota(jnp.int32, 16)
  iota8 = iota * _i32(8)
  col_ids = [jnp.full((16,), c, jnp.int32) for c in range(ENC_DIM)]
  zero16 = jnp.zeros((16,), jnp.int32)
  f_one = jnp.float32(1.0)
  f_zero = jnp.float32(0.0)

  def block_body(b, carry):
    rb = ray0 + b * _i32(B)
    pltpu.sync_copy(hist_hbm.at[pl.ds(rb * _i32(ENC_DEPTH), B * ENC_DEPTH)],
                    hist_v)
    pltpu.sync_copy(inp_hbm.at[pl.ds(rb, B)], inp_v)

    def gen_body(i, carry2):
      for g in range(NGRP):
        r0 = g * 16
        hv = plsc.load_gather(hist_v, [iota8 + (_i32(r0 * ENC_DEPTH) + i)])
        for j in range(8):
          plsc.store_scatter(
              tidx_v,
              [iota + ((i * _i32(8) + _i32(j)) * _i32(B) + _i32(r0))],
              hv ^ _i32(_HASH_C[j]))
      return carry2
    lax.fori_loop(_i32(0), _i32(ENC_DEPTH), gen_body, _i32(0))

    copies = []
    for c in range(B * 64 // CHUNK):
      copies.append(pltpu.async_copy(
          table_hbm.at[tidx_v.at[pl.ds(c * CHUNK, CHUNK)]],
          feat_v.at[pl.ds(c * CHUNK, CHUNK), :], sem))
    for c in range(NSLOT // CHUNK):
      sl = pl.ds(c * CHUNK, CHUNK)
      copies.append(pltpu.async_copy(
          nodes_hbm.at[hist_v.at[sl]], node_v.at[sl, :], sem))
    for cp in copies:
      cp.wait()

    for g in range(NGRP):
      r0 = g * 16
      rays = iota + _i32(r0)
      pts = [plsc.load_gather(inp_v, [rays, col_ids[p], col_ids[c]])
             for p in range(N_POINTS) for c in range(3)]

      def depth_body(i, carry2, r0=r0, rays=rays, pts=pts):
        pos8 = iota8 + (_i32(r0 * ENC_DEPTH) + i)
        mn = [plsc.load_gather(node_v, [pos8, col_ids[c]]) for c in range(3)]
        rex = [f_one / plsc.load_gather(node_v, [pos8, col_ids[3 + c]])
               for c in range(3)]
        ws = []
        for p in range(N_POINTS):
          xd, yd, zd = (
              jnp.clip((pts[p * 3 + c] - mn[c]) * rex[c], f_zero, f_one)
              for c in range(3))
          x0, y0, z0 = f_one - xd, f_one - yd, f_one - zd
          a00, a10, a01, a11 = x0 * y0, xd * y0, x0 * yd, xd * yd
          ws.append((a00 * z0, a10 * z0, a01 * z0, a00 * zd,
                     a10 * zd, a01 * zd, a11 * z0, a11 * zd))
        for d in range(ENC_DIM):
          fv = [plsc.load_gather(
                    feat_v,
                    [iota + ((i * _i32(8) + _i32(j)) * _i32(B) + _i32(r0)),
                     col_ids[d]])
                for j in range(8)]
          for p in range(N_POINTS):
            acc = ws[p][0] * fv[0]
            for j in range(1, 8):
              acc = acc + ws[p][j] * fv[j]
            plsc.store_scatter(
                out_v, [rays, zero16 + (i * _i32(32) + _i32(p * 8 + d))],
                acc)
        return carry2
      lax.fori_loop(_i32(0), _i32(ENC_DEPTH), depth_body, _i32(0))

    pltpu.sync_copy(out_v, out_hbm.at[pl.ds(rb, B), :])
    return carry
  lax.fori_loop(_i32(0), _i32(NBLK), block_body, _i32(0))


@functools.partial(
    pl.kernel,
    out_type=jax.ShapeDtypeStruct((N_RAYS, OUT_COLS), jnp.float32),
    mesh=plsc.VectorSubcoreMesh(core_axis_name="c", subcore_axis_name="s",
                                num_cores=NC, num_subcores=NS),
    scratch_types=[
        pltpu.VMEM((B * ENC_DEPTH,), jnp.int32),
        pltpu.VMEM((B * 64,), jnp.int32),
        pltpu.VMEM((B * 64, ENC_DIM), jnp.float32),
        pltpu.VMEM((NSLOT, 8), jnp.float32),
        pltpu.VMEM((B, N_POINTS, 3), jnp.float32),
        pltpu.VMEM((B, OUT_COLS), jnp.float32),
        pltpu.SemaphoreType.DMA,
    ],
    compiler_params=pltpu.CompilerParams(needs_layout_passes=False,
                                         use_tc_tiling_on_sc=False),
)
def _nbvh_sc(*refs):
  _sc_body(*refs)


def kernel(inp, history, nodes_min, nodes_extent, table):
  hist_i = history.astype(jnp.int32).reshape(-1)
  n = nodes_min.shape[0]
  nodes_cat = jnp.concatenate(
      [nodes_min.astype(jnp.float32), nodes_extent.astype(jnp.float32),
       jnp.zeros((n, 2), jnp.float32)], axis=1)
  return _nbvh_sc(inp.astype(jnp.float32), hist_i, nodes_cat,
                  table.astype(jnp.float32))


# speedup vs baseline: 9.4500x; 1.0739x over previous
import functools

import jax
import jax.numpy as jnp
from jax import lax
from jax.experimental import pallas as pl
from jax.experimental.pallas import tpu as pltpu
from jax.experimental.pallas import tpu_sc as plsc

N_RAYS = 65536
N_POINTS = 4
ENC_DIM = 8
ENC_DEPTH = 8
TABLE_SIZE = 2097152
N_NODES = 1000000
OUT_COLS = ENC_DEPTH * N_POINTS * ENC_DIM

_PIS = (774363409, 2654435761, 805459861, 100000007,
        334363391, 1334363413, 734363407, 2134363393)
_HASH_C = tuple(((j + 1) * _PIS[j]) % TABLE_SIZE for j in range(8))

NC, NS = 2, 16
NW = NC * NS
RPW = N_RAYS // NW
B = 64
NBLK = RPW // B
NGRP = B // 16
CHUNK = 128
NSLOT = B * ENC_DEPTH


def _i32(x):
  return jnp.int32(x)


def _sc_body(inp_hbm, hist_hbm, nodes_hbm, table_hbm, out_hbm,
             hist_v, tidx_v, feat_v, node_v, inp_v,
             hist2_v, tidx2_v, feat2_v, node2_v, inp2_v,
             out_v, semA, semB):
  wid = lax.axis_index("s") * _i32(NC) + lax.axis_index("c")
  ray0 = wid * _i32(RPW)
  iota = lax.iota(jnp.int32, 16)
  iota8 = iota * _i32(8)
  col_ids = [jnp.full((16,), c, jnp.int32) for c in range(ENC_DIM)]
  zero16 = jnp.zeros((16,), jnp.int32)
  f_one = jnp.float32(1.0)
  f_zero = jnp.float32(0.0)

  def stage(b, hist_b, tidx_b, feat_b, node_b, inp_b, sem):
    rb = ray0 + b * _i32(B)
    pltpu.sync_copy(hist_hbm.at[pl.ds(rb * _i32(ENC_DEPTH), B * ENC_DEPTH)],
                    hist_b)
    pltpu.sync_copy(inp_hbm.at[pl.ds(rb, B)], inp_b)

    def gen_body(i, carry2):
      for g in range(NGRP):
        r0 = g * 16
        hv = plsc.load_gather(hist_b, [iota8 + (_i32(r0 * ENC_DEPTH) + i)])
        for j in range(8):
          plsc.store_scatter(
              tidx_b,
              [iota + ((i * _i32(8) + _i32(j)) * _i32(B) + _i32(r0))],
              hv ^ _i32(_HASH_C[j]))
      return carry2
    lax.fori_loop(_i32(0), _i32(ENC_DEPTH), gen_body, _i32(0))

    for c in range(B * 64 // CHUNK):
      pltpu.async_copy(
          table_hbm.at[tidx_b.at[pl.ds(c * CHUNK, CHUNK)]],
          feat_b.at[pl.ds(c * CHUNK, CHUNK), :], sem)
    for c in range(NSLOT // CHUNK):
      sl = pl.ds(c * CHUNK, CHUNK)
      pltpu.async_copy(nodes_hbm.at[hist_b.at[sl]], node_b.at[sl, :], sem)

  def drain(tidx_b, feat_b, hist_b, node_b, sem):
    for c in range(B * 64 // CHUNK):
      pltpu.make_async_copy(
          table_hbm.at[tidx_b.at[pl.ds(c * CHUNK, CHUNK)]],
          feat_b.at[pl.ds(c * CHUNK, CHUNK), :], sem).wait()
    for c in range(NSLOT // CHUNK):
      sl = pl.ds(c * CHUNK, CHUNK)
      pltpu.make_async_copy(
          nodes_hbm.at[hist_b.at[sl]], node_b.at[sl, :], sem).wait()

  def compute(b, hist_b, feat_b, node_b, inp_b):
    rb = ray0 + b * _i32(B)
    for g in range(NGRP):
      r0 = g * 16
      rays = iota + _i32(r0)
      pts = [plsc.load_gather(inp_b, [rays, col_ids[p], col_ids[c]])
             for p in range(N_POINTS) for c in range(3)]

      def depth_body(i, carry2, r0=r0, rays=rays, pts=pts):
        pos8 = iota8 + (_i32(r0 * ENC_DEPTH) + i)
        mn = [plsc.load_gather(node_b, [pos8, col_ids[c]]) for c in range(3)]
        rex = [f_one / plsc.load_gather(node_b, [pos8, col_ids[3 + c]])
               for c in range(3)]
        ws = []
        for p in range(N_POINTS):
          xd, yd, zd = (
              jnp.clip((pts[p * 3 + c] - mn[c]) * rex[c], f_zero, f_one)
              for c in range(3))
          x0, y0, z0 = f_one - xd, f_one - yd, f_one - zd
          a00, a10, a01, a11 = x0 * y0, xd * y0, x0 * yd, xd * yd
          ws.append((a00 * z0, a10 * z0, a01 * z0, a00 * zd,
                     a10 * zd, a01 * zd, a11 * z0, a11 * zd))
        for d in range(ENC_DIM):
          fv = [plsc.load_gather(
                    feat_b,
                    [iota + ((i * _i32(8) + _i32(j)) * _i32(B) + _i32(r0)),
                     col_ids[d]])
                for j in range(8)]
          for p in range(N_POINTS):
            acc = ws[p][0] * fv[0]
            for j in range(1, 8):
              acc = acc + ws[p][j] * fv[j]
            plsc.store_scatter(
                out_v, [rays, zero16 + (i * _i32(32) + _i32(p * 8 + d))],
                acc)
        return carry2
      lax.fori_loop(_i32(0), _i32(ENC_DEPTH), depth_body, _i32(0))

    pltpu.sync_copy(out_v, out_hbm.at[pl.ds(rb, B), :])

  bufA = (hist_v, tidx_v, feat_v, node_v, inp_v, semA)
  bufB = (hist2_v, tidx2_v, feat2_v, node2_v, inp2_v, semB)

  def use(buf):
    hist_b, tidx_b, feat_b, node_b, inp_b, sem = buf
    return dict(stage=lambda b: stage(b, hist_b, tidx_b, feat_b, node_b,
                                      inp_b, sem),
                drain=lambda: drain(tidx_b, feat_b, hist_b, node_b, sem),
                compute=lambda b: compute(b, hist_b, feat_b, node_b, inp_b))

  A, Bf = use(bufA), use(bufB)
  A["stage"](_i32(0))

  def pair_body(k, carry):
    b = k * _i32(2)
    Bf["stage"](b + _i32(1))
    A["drain"]()
    A["compute"](b)

    @pl.when(b + _i32(2) < _i32(NBLK))
    def _():
      A["stage"](b + _i32(2))
    Bf["drain"]()
    Bf["compute"](b + _i32(1))
    return carry
  lax.fori_loop(_i32(0), _i32(NBLK // 2), pair_body, _i32(0))


@functools.partial(
    pl.kernel,
    out_type=jax.ShapeDtypeStruct((N_RAYS, OUT_COLS), jnp.float32),
    mesh=plsc.VectorSubcoreMesh(core_axis_name="c", subcore_axis_name="s",
                                num_cores=NC, num_subcores=NS),
    scratch_types=[
        pltpu.VMEM((B * ENC_DEPTH,), jnp.int32),
        pltpu.VMEM((B * 64,), jnp.int32),
        pltpu.VMEM((B * 64, ENC_DIM), jnp.float32),
        pltpu.VMEM((NSLOT, 8), jnp.float32),
        pltpu.VMEM((B, N_POINTS, 3), jnp.float32),
        pltpu.VMEM((B * ENC_DEPTH,), jnp.int32),
        pltpu.VMEM((B * 64,), jnp.int32),
        pltpu.VMEM((B * 64, ENC_DIM), jnp.float32),
        pltpu.VMEM((NSLOT, 8), jnp.float32),
        pltpu.VMEM((B, N_POINTS, 3), jnp.float32),
        pltpu.VMEM((B, OUT_COLS), jnp.float32),
        pltpu.SemaphoreType.DMA,
        pltpu.SemaphoreType.DMA,
    ],
    compiler_params=pltpu.CompilerParams(needs_layout_passes=False,
                                         use_tc_tiling_on_sc=False),
)
def _nbvh_sc(*refs):
  _sc_body(*refs)


def kernel(inp, history, nodes_min, nodes_extent, table):
  hist_i = history.astype(jnp.int32).reshape(-1)
  n = nodes_min.shape[0]
  nodes_cat = jnp.concatenate(
      [nodes_min.astype(jnp.float32), nodes_extent.astype(jnp.float32),
       jnp.zeros((n, 2), jnp.float32)], axis=1)
  return _nbvh_sc(inp.astype(jnp.float32), hist_i, nodes_cat,
                  table.astype(jnp.float32))


# speedup vs baseline: 9.5061x; 1.0059x over previous
import functools

import jax
import jax.numpy as jnp
from jax import lax
from jax.experimental import pallas as pl
from jax.experimental.pallas import tpu as pltpu
from jax.experimental.pallas import tpu_sc as plsc

N_RAYS = 65536
N_POINTS = 4
ENC_DIM = 8
ENC_DEPTH = 8
TABLE_SIZE = 2097152
N_NODES = 1000000
OUT_COLS = ENC_DEPTH * N_POINTS * ENC_DIM

_PIS = (774363409, 2654435761, 805459861, 100000007,
        334363391, 1334363413, 734363407, 2134363393)
_HASH_C = tuple(((j + 1) * _PIS[j]) % TABLE_SIZE for j in range(8))

NC, NS = 2, 16
NW = NC * NS
RPW = N_RAYS // NW
B = 64
NBLK = RPW // B
NGRP = B // 16
CHUNK = 128
NSLOT = B * ENC_DEPTH


def _i32(x):
  return jnp.int32(x)


def _sc_body(inp_hbm, hist_hbm, nodes_hbm, table_hbm, out_hbm,
             hist_v, tidx_v, feat_v, node_v, inp_v,
             hist2_v, tidx2_v, feat2_v, node2_v, inp2_v,
             out_v, out2_v, semA, semB, semOA, semOB):
  wid = lax.axis_index("s") * _i32(NC) + lax.axis_index("c")
  ray0 = wid * _i32(RPW)
  iota = lax.iota(jnp.int32, 16)
  iota8 = iota * _i32(8)
  col_ids = [jnp.full((16,), c, jnp.int32) for c in range(ENC_DIM)]
  zero16 = jnp.zeros((16,), jnp.int32)
  f_one = jnp.float32(1.0)
  f_zero = jnp.float32(0.0)

  def stage(b, hist_b, tidx_b, feat_b, node_b, inp_b, sem):
    rb = ray0 + b * _i32(B)
    pltpu.sync_copy(hist_hbm.at[pl.ds(rb * _i32(ENC_DEPTH), B * ENC_DEPTH)],
                    hist_b)
    pltpu.sync_copy(inp_hbm.at[pl.ds(rb, B)], inp_b)

    def gen_body(i, carry2):
      for g in range(NGRP):
        r0 = g * 16
        hv = plsc.load_gather(hist_b, [iota8 + (_i32(r0 * ENC_DEPTH) + i)])
        for j in range(8):
          plsc.store_scatter(
              tidx_b,
              [iota + ((i * _i32(8) + _i32(j)) * _i32(B) + _i32(r0))],
              hv ^ _i32(_HASH_C[j]))
      return carry2
    lax.fori_loop(_i32(0), _i32(ENC_DEPTH), gen_body, _i32(0))

    for c in range(B * 64 // CHUNK):
      pltpu.async_copy(
          table_hbm.at[tidx_b.at[pl.ds(c * CHUNK, CHUNK)]],
          feat_b.at[pl.ds(c * CHUNK, CHUNK), :], sem)
    for c in range(NSLOT // CHUNK):
      sl = pl.ds(c * CHUNK, CHUNK)
      pltpu.async_copy(nodes_hbm.at[hist_b.at[sl]], node_b.at[sl, :], sem)

  def drain(tidx_b, feat_b, hist_b, node_b, sem):
    for c in range(B * 64 // CHUNK):
      pltpu.make_async_copy(
          table_hbm.at[tidx_b.at[pl.ds(c * CHUNK, CHUNK)]],
          feat_b.at[pl.ds(c * CHUNK, CHUNK), :], sem).wait()
    for c in range(NSLOT // CHUNK):
      sl = pl.ds(c * CHUNK, CHUNK)
      pltpu.make_async_copy(
          nodes_hbm.at[hist_b.at[sl]], node_b.at[sl, :], sem).wait()

  def compute(b, first, hist_b, feat_b, node_b, inp_b, out_b, semO):
    rb = ray0 + b * _i32(B)

    @pl.when(jnp.logical_not(first))
    def _():
      pltpu.make_async_copy(
          out_b, out_hbm.at[pl.ds(ray0, B), :], semO).wait()

    for g in range(NGRP):
      r0 = g * 16
      rays = iota + _i32(r0)
      pts = [plsc.load_gather(inp_b, [rays, col_ids[p], col_ids[c]])
             for p in range(N_POINTS) for c in range(3)]

      def depth_body(i, carry2, r0=r0, rays=rays, pts=pts):
        pos8 = iota8 + (_i32(r0 * ENC_DEPTH) + i)
        mn = [plsc.load_gather(node_b, [pos8, col_ids[c]]) for c in range(3)]
        rex = [f_one / plsc.load_gather(node_b, [pos8, col_ids[3 + c]])
               for c in range(3)]
        ws = []
        for p in range(N_POINTS):
          xd, yd, zd = (
              jnp.clip((pts[p * 3 + c] - mn[c]) * rex[c], f_zero, f_one)
              for c in range(3))
          x0, y0, z0 = f_one - xd, f_one - yd, f_one - zd
          a00, a10, a01, a11 = x0 * y0, xd * y0, x0 * yd, xd * yd
          ws.append((a00 * z0, a10 * z0, a01 * z0, a00 * zd,
                     a10 * zd, a01 * zd, a11 * z0, a11 * zd))
        for d in range(ENC_DIM):
          fv = [plsc.load_gather(
                    feat_b,
                    [iota + ((i * _i32(8) + _i32(j)) * _i32(B) + _i32(r0)),
                     col_ids[d]])
                for j in range(8)]
          for p in range(N_POINTS):
            acc = ws[p][0] * fv[0]
            for j in range(1, 8):
              acc = acc + ws[p][j] * fv[j]
            plsc.store_scatter(
                out_b, [rays, zero16 + (i * _i32(32) + _i32(p * 8 + d))],
                acc)
        return carry2
      lax.fori_loop(_i32(0), _i32(ENC_DEPTH), depth_body, _i32(0))

    pltpu.async_copy(out_b, out_hbm.at[pl.ds(rb, B), :], semO)

  bufA = (hist_v, tidx_v, feat_v, node_v, inp_v, out_v, semA, semOA)
  bufB = (hist2_v, tidx2_v, feat2_v, node2_v, inp2_v, out2_v, semB, semOB)

  def use(buf):
    hist_b, tidx_b, feat_b, node_b, inp_b, out_b, sem, semO = buf
    return dict(stage=lambda b: stage(b, hist_b, tidx_b, feat_b, node_b,
                                      inp_b, sem),
                drain=lambda: drain(tidx_b, feat_b, hist_b, node_b, sem),
                compute=lambda b, first: compute(b, first, hist_b, feat_b,
                                                 node_b, inp_b, out_b, semO),
                finish=lambda: pltpu.make_async_copy(
                    out_b, out_hbm.at[pl.ds(ray0, B), :], semO).wait())

  A, Bf = use(bufA), use(bufB)
  A["stage"](_i32(0))

  def pair_body(k, carry):
    b = k * _i32(2)
    first = k == _i32(0)
    Bf["stage"](b + _i32(1))
    A["drain"]()
    A["compute"](b, first)

    @pl.when(b + _i32(2) < _i32(NBLK))
    def _():
      A["stage"](b + _i32(2))
    Bf["drain"]()
    Bf["compute"](b + _i32(1), first)
    return carry
  lax.fori_loop(_i32(0), _i32(NBLK // 2), pair_body, _i32(0))
  A["finish"]()
  Bf["finish"]()


@functools.partial(
    pl.kernel,
    out_type=jax.ShapeDtypeStruct((N_RAYS, OUT_COLS), jnp.float32),
    mesh=plsc.VectorSubcoreMesh(core_axis_name="c", subcore_axis_name="s",
                                num_cores=NC, num_subcores=NS),
    scratch_types=[
        pltpu.VMEM((B * ENC_DEPTH,), jnp.int32),
        pltpu.VMEM((B * 64,), jnp.int32),
        pltpu.VMEM((B * 64, ENC_DIM), jnp.float32),
        pltpu.VMEM((NSLOT, 8), jnp.float32),
        pltpu.VMEM((B, N_POINTS, 3), jnp.float32),
        pltpu.VMEM((B * ENC_DEPTH,), jnp.int32),
        pltpu.VMEM((B * 64,), jnp.int32),
        pltpu.VMEM((B * 64, ENC_DIM), jnp.float32),
        pltpu.VMEM((NSLOT, 8), jnp.float32),
        pltpu.VMEM((B, N_POINTS, 3), jnp.float32),
        pltpu.VMEM((B, OUT_COLS), jnp.float32),
        pltpu.VMEM((B, OUT_COLS), jnp.float32),
        pltpu.SemaphoreType.DMA,
        pltpu.SemaphoreType.DMA,
        pltpu.SemaphoreType.DMA,
        pltpu.SemaphoreType.DMA,
    ],
    compiler_params=pltpu.CompilerParams(needs_layout_passes=False,
                                         use_tc_tiling_on_sc=False),
)
def _nbvh_sc(*refs):
  _sc_body(*refs)


def kernel(inp, history, nodes_min, nodes_extent, table):
  hist_i = history.astype(jnp.int32).reshape(-1)
  n = nodes_min.shape[0]
  nodes_cat = jnp.concatenate(
      [nodes_min.astype(jnp.float32), nodes_extent.astype(jnp.float32),
       jnp.zeros((n, 2), jnp.float32)], axis=1)
  return _nbvh_sc(inp.astype(jnp.float32), hist_i, nodes_cat,
                  table.astype(jnp.float32))
